# baseline (speedup 1.0000x reference)
import functools

import jax
import jax.numpy as jnp
from jax import lax
from jax.experimental import pallas as pl
from jax.experimental.pallas import tpu as pltpu
from jax.experimental.pallas import tpu_sc as plsc

N_BINS = 15
NC = 2
NS = 16
NW = NC * NS
LANES = 16
HSLOTS = N_BINS * LANES
NCOPY = 8
HPAD = (N_BINS + 1) * LANES

TC_COLS = 1024
TC_ROWS = 512
TC_ACC_ROWS = N_BINS * 8


def _sc_hist_call(n_sc):
    per_w = n_sc // NW
    ch = 16384
    assert per_w % (2 * ch) == 0
    n_chunks = per_w // ch
    n_vecs = ch // LANES

    mesh = plsc.VectorSubcoreMesh(
        core_axis_name="c", subcore_axis_name="s",
        num_cores=NC, num_subcores=NS)

    @functools.partial(
        pl.kernel,
        out_type=jax.ShapeDtypeStruct((NW, HSLOTS), jnp.float32),
        mesh=mesh,
        scratch_types=[
            pltpu.VMEM((ch,), jnp.float32),
            pltpu.VMEM((ch,), jnp.float32),
            pltpu.VMEM((ch,), jnp.int32),
            pltpu.VMEM((ch,), jnp.int32),
            pltpu.VMEM((NCOPY * HPAD,), jnp.float32),
            pltpu.VMEM((HSLOTS,), jnp.float32),
            pltpu.SemaphoreType.DMA,
            pltpu.SemaphoreType.DMA,
        ],
        compiler_params=pltpu.CompilerParams(
            needs_layout_passes=False, disable_bounds_checks=True),
    )
    def sc_hist(log_hbm, lab_hbm, outf_hbm,
                lb0, lb1, ab0, ab1, hist_f, hout_f, sem0, sem1):
        wid = lax.axis_index("s") * NC + lax.axis_index("c")
        base = wid * per_w
        lane = lax.iota(jnp.int32, 16)
        lanes_k = [lane + k * HPAD for k in range(NCOPY)]
        lbufs, abufs, sems = (lb0, lb1), (ab0, ab1), (sem0, sem1)

        for r in range((N_BINS + 1) * NCOPY):
            hist_f[pl.ds(r * LANES, LANES)] = jnp.zeros((LANES,), jnp.float32)

        def start(g, b):
            pltpu.async_copy(log_hbm.at[pl.ds(base + g * ch, ch)], lbufs[b], sems[b])
            pltpu.async_copy(lab_hbm.at[pl.ds(base + g * ch, ch)], abufs[b], sems[b])

        def wait(b):
            pltpu.make_async_copy(log_hbm.at[pl.ds(0, ch)], lbufs[b], sems[b]).wait()
            pltpu.make_async_copy(lab_hbm.at[pl.ds(0, ch)], abufs[b], sems[b]).wait()

        def compute(b):
            lbuf, abuf = lbufs[b], abufs[b]

            def group_body(i, carry):
                for k in range(NCOPY):
                    off = i * (LANES * NCOPY) + k * LANES
                    x = lbuf[pl.ds(off, LANES)]
                    lab = abuf[pl.ds(off, LANES)]
                    conf = 1.0 / (1.0 + jnp.exp(-x))
                    idx = jnp.minimum((conf * 15.0).astype(jnp.int32), 14)
                    idx = jnp.where(conf > 0.0, idx, N_BINS)
                    addr = idx * LANES + lanes_k[k]
                    plsc.addupdate_scatter(hist_f, [addr],
                                           conf - lab.astype(jnp.float32))
                return carry

            lax.fori_loop(0, n_vecs // NCOPY, group_body, None)

        start(0, 0)
        start(1, 1)

        def pair_body(p, carry):
            for b in range(2):
                g = p * 2 + b
                wait(b)
                compute(b)

                @pl.when(g + 2 < n_chunks)
                def _():
                    start(g + 2, b)
            return carry

        lax.fori_loop(0, n_chunks // 2, pair_body, None)

        for r in range(N_BINS):
            acc_f = hist_f[pl.ds(r * LANES, LANES)]
            for k in range(1, NCOPY):
                acc_f = acc_f + hist_f[pl.ds(k * HPAD + r * LANES, LANES)]
            hout_f[pl.ds(r * LANES, LANES)] = acc_f
        pltpu.sync_copy(hout_f, outf_hbm.at[wid])

    return sc_hist


def _tc_hist_body(x_ref, l_ref, o_ref):
    g = pl.program_id(0)

    @pl.when(g == 0)
    def _():
        o_ref[...] = jnp.zeros((TC_ACC_ROWS, TC_COLS), jnp.float32)

    x = x_ref[...]
    conf = 1.0 / (1.0 + jnp.exp(-x))
    idx = jnp.minimum((conf * 15.0).astype(jnp.int32), 14)
    idx = jnp.where(conf > 0.0, idx, N_BINS)
    v = conf - l_ref[...].astype(jnp.float32)
    for b in range(N_BINS):
        mv = jnp.where(idx == b, v, 0.0)
        acc = mv.reshape(TC_ROWS // 8, 8, TC_COLS).sum(axis=0)
        o_ref[pl.ds(b * 8, 8), :] += acc


def _finalize_body(n, ff_ref, tc_ref, o_ref):
    d = jnp.sum(ff_ref[...], axis=0, keepdims=True)
    r = lax.broadcasted_iota(jnp.int32, (HSLOTS, N_BINS), 0) // LANES
    c = lax.broadcasted_iota(jnp.int32, (HSLOTS, N_BINS), 1)
    m = (r == c).astype(jnp.float32)
    dot = functools.partial(jnp.dot, precision=lax.Precision.HIGHEST)
    d_b = dot(d, m)

    tc = tc_ref[...]
    tc_s = dot(tc, jnp.ones((TC_COLS, 1), jnp.float32))
    rr = lax.broadcasted_iota(jnp.int32, (N_BINS, TC_ACC_ROWS), 1) // 8
    bb = lax.broadcasted_iota(jnp.int32, (N_BINS, TC_ACC_ROWS), 0)
    eye = (lax.broadcasted_iota(jnp.int32, (N_BINS, N_BINS), 0)
           == lax.broadcasted_iota(jnp.int32, (N_BINS, N_BINS), 1)).astype(jnp.float32)
    sel = (rr == bb).astype(jnp.float32)
    col = lax.dot_general(
        sel, tc_s, (((1,), (0,)), ((), ())),
        precision=lax.Precision.HIGHEST)
    d_b = d_b + lax.dot_general(
        col, eye, (((0,), (0,)), ((), ())),
        precision=lax.Precision.HIGHEST)

    o_ref[...] = jnp.sum(jnp.abs(d_b), axis=1, keepdims=True) / n


def kernel(logits, labels):
    n = logits.shape[0]
    labels = labels.astype(jnp.int32)

    n_sc = 5 * 1048576
    n_tc = n - n_sc
    assert n_tc % (TC_ROWS * TC_COLS) == 0
    grid = n_tc // (TC_ROWS * TC_COLS)
    off = n_sc // (TC_ROWS * TC_COLS)

    outf = _sc_hist_call(n_sc)(logits, labels)

    x2 = logits.reshape(n // TC_COLS, TC_COLS)
    l2 = labels.reshape(n // TC_COLS, TC_COLS)
    tc_part = pl.pallas_call(
        _tc_hist_body,
        grid=(grid,),
        in_specs=[
            pl.BlockSpec((TC_ROWS, TC_COLS), lambda i: (i + off, 0)),
            pl.BlockSpec((TC_ROWS, TC_COLS), lambda i: (i + off, 0)),
        ],
        out_specs=pl.BlockSpec((TC_ACC_ROWS, TC_COLS), lambda i: (0, 0)),
        out_shape=jax.ShapeDtypeStruct((TC_ACC_ROWS, TC_COLS), jnp.float32),
    )(x2, l2)

    ece = pl.pallas_call(
        functools.partial(_finalize_body, n),
        out_shape=jax.ShapeDtypeStruct((1, 1), jnp.float32),
    )(outf, tc_part)
    return ece.reshape(1)


# speedup vs baseline: 1.1861x; 1.1861x over previous
import functools

import jax
import jax.numpy as jnp
from jax import lax
from jax.experimental import pallas as pl
from jax.experimental.pallas import tpu as pltpu
from jax.experimental.pallas import tpu_sc as plsc

N_BINS = 15
NC = 2
NS = 16
NW = NC * NS
LANES = 16
HSLOTS = N_BINS * LANES
NCOPY = 8
HPAD = (N_BINS + 1) * LANES

TC_COLS = 1024
TC_ROWS = 512
TC_ACC_ROWS = N_BINS * 8


def _sc_hist_call(n_sc):
    per_w = n_sc // NW
    ch = 16384
    assert per_w % (2 * ch) == 0
    n_chunks = per_w // ch
    n_vecs = ch // LANES

    mesh = plsc.VectorSubcoreMesh(
        core_axis_name="c", subcore_axis_name="s",
        num_cores=NC, num_subcores=NS)

    @functools.partial(
        pl.kernel,
        out_type=jax.ShapeDtypeStruct((NW, HSLOTS), jnp.float32),
        mesh=mesh,
        scratch_types=[
            pltpu.VMEM((ch,), jnp.float32),
            pltpu.VMEM((ch,), jnp.float32),
            pltpu.VMEM((ch,), jnp.int32),
            pltpu.VMEM((ch,), jnp.int32),
            pltpu.VMEM((NCOPY * HPAD,), jnp.float32),
            pltpu.VMEM((HSLOTS,), jnp.float32),
            pltpu.SemaphoreType.DMA,
            pltpu.SemaphoreType.DMA,
        ],
        compiler_params=pltpu.CompilerParams(
            needs_layout_passes=False, disable_bounds_checks=True),
    )
    def sc_hist(log_hbm, lab_hbm, outf_hbm,
                lb0, lb1, ab0, ab1, hist_f, hout_f, sem0, sem1):
        wid = lax.axis_index("s") * NC + lax.axis_index("c")
        base = wid * per_w
        lane = lax.iota(jnp.int32, 16)
        lanes_k = [lane + k * HPAD for k in range(NCOPY)]
        lbufs, abufs, sems = (lb0, lb1), (ab0, ab1), (sem0, sem1)

        for r in range((N_BINS + 1) * NCOPY):
            hist_f[pl.ds(r * LANES, LANES)] = jnp.zeros((LANES,), jnp.float32)

        def start(g, b):
            pltpu.async_copy(log_hbm.at[pl.ds(base + g * ch, ch)], lbufs[b], sems[b])
            pltpu.async_copy(lab_hbm.at[pl.ds(base + g * ch, ch)], abufs[b], sems[b])

        def wait(b):
            pltpu.make_async_copy(log_hbm.at[pl.ds(0, ch)], lbufs[b], sems[b]).wait()
            pltpu.make_async_copy(lab_hbm.at[pl.ds(0, ch)], abufs[b], sems[b]).wait()

        def compute(b):
            lbuf, abuf = lbufs[b], abufs[b]

            def group_body(i, carry):
                for k in range(NCOPY):
                    off = i * (LANES * NCOPY) + k * LANES
                    x = lbuf[pl.ds(off, LANES)]
                    lab = abuf[pl.ds(off, LANES)]
                    conf = 1.0 / (1.0 + jnp.exp(-x))
                    idx = jnp.minimum((conf * 15.0).astype(jnp.int32), 14)
                    idx = jnp.where(conf > 0.0, idx, N_BINS)
                    addr = idx * LANES + lanes_k[k]
                    plsc.addupdate_scatter(hist_f, [addr],
                                           conf - lab.astype(jnp.float32))
                return carry

            lax.fori_loop(0, n_vecs // NCOPY, group_body, None)

        start(0, 0)
        start(1, 1)

        def pair_body(p, carry):
            for b in range(2):
                g = p * 2 + b
                wait(b)
                compute(b)

                @pl.when(g + 2 < n_chunks)
                def _():
                    start(g + 2, b)
            return carry

        lax.fori_loop(0, n_chunks // 2, pair_body, None)

        for r in range(N_BINS):
            acc_f = hist_f[pl.ds(r * LANES, LANES)]
            for k in range(1, NCOPY):
                acc_f = acc_f + hist_f[pl.ds(k * HPAD + r * LANES, LANES)]
            hout_f[pl.ds(r * LANES, LANES)] = acc_f
        pltpu.sync_copy(hout_f, outf_hbm.at[wid])

    return sc_hist


def _tc_hist_body(x_ref, l_ref, o_ref):
    g = pl.program_id(0)

    @pl.when(g == 0)
    def _():
        o_ref[...] = jnp.zeros((TC_ACC_ROWS, TC_COLS), jnp.float32)

    x = x_ref[...]
    conf = 1.0 / (1.0 + jnp.exp(-x))
    idx = jnp.minimum((conf * 15.0).astype(jnp.int32), 14)
    idx = jnp.where(conf > 0.0, idx, N_BINS)
    v = conf - l_ref[...].astype(jnp.float32)
    for b in range(N_BINS):
        mv = jnp.where(idx == b, v, 0.0)
        acc = mv.reshape(TC_ROWS // 8, 8, TC_COLS).sum(axis=0)
        o_ref[pl.ds(b * 8, 8), :] += acc


def _finalize_body(n, ff_ref, tc_ref, o_ref):
    d = jnp.sum(ff_ref[...], axis=0, keepdims=True)
    r = lax.broadcasted_iota(jnp.int32, (HSLOTS, N_BINS), 0) // LANES
    c = lax.broadcasted_iota(jnp.int32, (HSLOTS, N_BINS), 1)
    m = (r == c).astype(jnp.float32)
    dot = functools.partial(jnp.dot, precision=lax.Precision.HIGHEST)
    d_b = dot(d, m)

    tc = tc_ref[...]
    tc_s = dot(tc, jnp.ones((TC_COLS, 1), jnp.float32))
    rr = lax.broadcasted_iota(jnp.int32, (N_BINS, TC_ACC_ROWS), 1) // 8
    bb = lax.broadcasted_iota(jnp.int32, (N_BINS, TC_ACC_ROWS), 0)
    eye = (lax.broadcasted_iota(jnp.int32, (N_BINS, N_BINS), 0)
           == lax.broadcasted_iota(jnp.int32, (N_BINS, N_BINS), 1)).astype(jnp.float32)
    sel = (rr == bb).astype(jnp.float32)
    col = lax.dot_general(
        sel, tc_s, (((1,), (0,)), ((), ())),
        precision=lax.Precision.HIGHEST)
    d_b = d_b + lax.dot_general(
        col, eye, (((0,), (0,)), ((), ())),
        precision=lax.Precision.HIGHEST)

    o_ref[...] = jnp.sum(jnp.abs(d_b), axis=1, keepdims=True) / n


def kernel(logits, labels):
    n = logits.shape[0]
    labels = labels.astype(jnp.int32)

    n_sc = 4 * 1048576
    n_tc = n - n_sc
    assert n_tc % (TC_ROWS * TC_COLS) == 0
    grid = n_tc // (TC_ROWS * TC_COLS)
    off = n_sc // (TC_ROWS * TC_COLS)

    outf = _sc_hist_call(n_sc)(logits, labels)

    x2 = logits.reshape(n // TC_COLS, TC_COLS)
    l2 = labels.reshape(n // TC_COLS, TC_COLS)
    tc_part = pl.pallas_call(
        _tc_hist_body,
        grid=(grid,),
        in_specs=[
            pl.BlockSpec((TC_ROWS, TC_COLS), lambda i: (i + off, 0)),
            pl.BlockSpec((TC_ROWS, TC_COLS), lambda i: (i + off, 0)),
        ],
        out_specs=pl.BlockSpec((TC_ACC_ROWS, TC_COLS), lambda i: (0, 0)),
        out_shape=jax.ShapeDtypeStruct((TC_ACC_ROWS, TC_COLS), jnp.float32),
    )(x2, l2)

    ece = pl.pallas_call(
        functools.partial(_finalize_body, n),
        out_shape=jax.ShapeDtypeStruct((1, 1), jnp.float32),
    )(outf, tc_part)
    return ece.reshape(1)
